# baseline (speedup 1.0000x reference)
import functools

import jax
import jax.numpy as jnp
from jax import lax
from jax.experimental import pallas as pl
from jax.experimental.pallas import tpu as pltpu
from jax.experimental.pallas import tpu_sc as plsc

_LEAK = 0.01
_F32 = jnp.float32
_I32 = jnp.int32
_NC, _NS, _NW = 2, 16, 32


def _lrelu(x):
    return jnp.maximum(x, _LEAK * x)


def _pad_rows(a, n):
    return jnp.pad(a, ((0, n - a.shape[0]), (0, 0)))


def _pad1(a, n, val):
    a = a.astype(_I32)
    return jnp.concatenate([a, jnp.full((n - a.shape[0],), val, _I32)])


def _eparts(e_list, W):
    parts, off = [], 0
    for ep, kreal in e_list:
        wpart = W[off:off + kreal]
        wpart = jnp.pad(wpart, ((0, ep.shape[1] - kreal), (0, 0)))
        parts.append((ep, wpart))
        off += kreal
    return parts


def _wpadc(W, w):
    return jnp.pad(W, ((0, 0), (0, w - W.shape[1])))


def _tc_mm(parts, b, act, m_real, bm=256):
    M = parts[0][0].shape[0]
    N = parts[0][1].shape[1]
    nP = len(parts)

    def kern(*refs):
        o_ref = refs[-1]
        acc = jnp.zeros((bm, N), _F32) + refs[2 * nP][...]
        for i in range(nP):
            acc = acc + jnp.dot(refs[2 * i][...], refs[2 * i + 1][...],
                                preferred_element_type=_F32)
        if act:
            acc = _lrelu(acc)
        if m_real is not None:
            rows = pl.program_id(0) * bm + lax.broadcasted_iota(_I32, (bm, N), 0)
            acc = jnp.where(rows < m_real, acc, 0.0)
        o_ref[...] = acc

    in_specs = []
    args = []
    for X, W in parts:
        K = X.shape[1]
        in_specs.append(pl.BlockSpec((bm, K), lambda i: (i, 0)))
        in_specs.append(pl.BlockSpec((K, N), lambda i: (0, 0)))
        args += [X, W]
    in_specs.append(pl.BlockSpec((1, N), lambda i: (0, 0)))
    args.append(b.reshape(1, N))
    return pl.pallas_call(
        kern, grid=(M // bm,), in_specs=in_specs,
        out_specs=pl.BlockSpec((bm, N), lambda i: (i, 0)),
        out_shape=jax.ShapeDtypeStruct((M, N), _F32))(*args)


def _tc_head(xparts, W1, b1, W2, b2, g, bb, bm=256):
    M = xparts[0][0].shape[0]
    H = W1.shape[1]
    NO = W2.shape[1]
    parts = _eparts(xparts, W1)
    nP = len(parts)

    def kern(*refs):
        o_ref = refs[-1]
        t = jnp.zeros((bm, H), _F32) + refs[2 * nP][...]
        for i in range(nP):
            t = t + jnp.dot(refs[2 * i][...], refs[2 * i + 1][...],
                            preferred_element_type=_F32)
        t = _lrelu(t)
        u = jnp.dot(t, refs[2 * nP + 1][...],
                    preferred_element_type=_F32) + refs[2 * nP + 2][...]
        mu = jnp.mean(u, axis=-1, keepdims=True)
        var = jnp.mean((u - mu) ** 2, axis=-1, keepdims=True)
        o_ref[...] = ((u - mu) * lax.rsqrt(var + 1e-5) * refs[2 * nP + 3][...]
                      + refs[2 * nP + 4][...])

    in_specs = []
    args = []
    for X, W in parts:
        K = X.shape[1]
        in_specs.append(pl.BlockSpec((bm, K), lambda i: (i, 0)))
        in_specs.append(pl.BlockSpec((K, H), lambda i: (0, 0)))
        args += [X, W]
    in_specs += [pl.BlockSpec((1, H), lambda i: (0, 0)),
                 pl.BlockSpec((H, NO), lambda i: (0, 0)),
                 pl.BlockSpec((1, NO), lambda i: (0, 0)),
                 pl.BlockSpec((1, NO), lambda i: (0, 0)),
                 pl.BlockSpec((1, NO), lambda i: (0, 0))]
    args += [b1.reshape(1, H), W2, b2.reshape(1, NO), g.reshape(1, NO),
             bb.reshape(1, NO)]
    return pl.pallas_call(
        kern, grid=(M // bm,), in_specs=in_specs,
        out_specs=pl.BlockSpec((bm, NO), lambda i: (i, 0)),
        out_shape=jax.ShapeDtypeStruct((M, NO), _F32))(*args)


def _bcast(v16, l):
    return v16.at[jnp.full((16,), l, _I32)].get(mode='promise_in_bounds')


def _sc_edge(A0, A1, B0, B1, C0, C1, srcg, dstg, eg, be0, be1, valid, *,
             dh, Ep, CH, gather_c):
    EPT = Ep // _NS
    nch = EPT // CH
    dw = dh // 16
    mesh = plsc.VectorSubcoreMesh(core_axis_name="c", subcore_axis_name="s")
    out_type = (jax.ShapeDtypeStruct((Ep, dh), _F32),
                jax.ShapeDtypeStruct((Ep, dh), _F32))
    scratch = [
        pltpu.VMEM((CH, dh), _F32), pltpu.VMEM((CH, dh), _F32),
        pltpu.VMEM((CH, dh), _F32),
        pltpu.VMEM((CH,), _I32), pltpu.VMEM((CH,), _I32),
        pltpu.VMEM((CH,), _I32), pltpu.VMEM((CH,), _F32),
        pltpu.VMEM((dh,), _F32),
        pltpu.SemaphoreType.DMA, pltpu.SemaphoreType.DMA,
        pltpu.SemaphoreType.DMA,
    ]

    @functools.partial(pl.kernel, out_type=out_type, mesh=mesh,
                       scratch_types=scratch)
    def run(A0_h, A1_h, B0_h, B1_h, C0_h, C1_h, sg_h, dg_h, eg_h,
            be0_h, be1_h, vl_h, e0_h, e1_h,
            abuf, bbuf, cbuf, si, di, ei, vbuf, biasv, s1, s2, s3):
        c = lax.axis_index("c")
        s = lax.axis_index("s")

        def work(A_h, B_h, C_h, be_h, enew_h):
            pltpu.sync_copy(be_h, biasv)

            @pl.loop(0, nch)
            def _chunk(ci):
                base = s * EPT + ci * CH
                pltpu.sync_copy(sg_h.at[pl.ds(base, CH)], si)
                pltpu.sync_copy(dg_h.at[pl.ds(base, CH)], di)
                pltpu.sync_copy(vl_h.at[pl.ds(base, CH)], vbuf)
                if gather_c:
                    pltpu.sync_copy(eg_h.at[pl.ds(base, CH)], ei)
                    h3 = pltpu.async_copy(C_h.at[ei], cbuf, s3)
                else:
                    h3 = pltpu.async_copy(C_h.at[pl.ds(base, CH)], cbuf, s3)
                h1 = pltpu.async_copy(A_h.at[si], abuf, s1)
                h2 = pltpu.async_copy(B_h.at[di], bbuf, s2)
                h1.wait()
                h2.wait()
                h3.wait()

                @pl.loop(0, CH // 16)
                def _grp(g):
                    vv16 = vbuf[pl.ds(g * 16, 16)]
                    for l in range(16):
                        r = g * 16 + l
                        vb = _bcast(vv16, l)
                        for j in range(dw):
                            sl = pl.ds(j * 16, 16)
                            v = (abuf[r, sl] + bbuf[r, sl] + cbuf[r, sl]
                                 + biasv[sl])
                            abuf[r, sl] = jnp.maximum(v, v * _LEAK) * vb

                pltpu.sync_copy(abuf, enew_h.at[pl.ds(base, CH)])

        @pl.when(c == 0)
        def _c0():
            work(A0_h, B0_h, C0_h, be0_h, e0_h)

        @pl.when(c == 1)
        def _c1():
            work(A1_h, B1_h, C1_h, be1_h, e1_h)

    return run(A0, A1, B0, B1, C0, C1, srcg, dstg, eg, be0, be1, valid)


def _sc_scan(e0, e1, sep, keep, *, dh, Ep, CH):
    EPT = Ep // _NS
    nch = EPT // CH
    dw = dh // 16
    mesh = plsc.VectorSubcoreMesh(core_axis_name="c", subcore_axis_name="s")
    out_type = (jax.ShapeDtypeStruct((Ep, dh), _F32),
                jax.ShapeDtypeStruct((Ep, dh), _F32))
    scratch = [
        pltpu.VMEM((CH, dh), _F32),
        pltpu.VMEM((CH,), _I32),
        pltpu.VMEM((CH,), _F32),
        pltpu.SemaphoreType.DMA,
    ]

    @functools.partial(pl.kernel, out_type=out_type, mesh=mesh,
                       scratch_types=scratch)
    def run(e0_h, e1_h, sep_h, kp_h, o0_h, o1_h, ebuf, sei, kbuf, s1):
        c = lax.axis_index("c")
        s = lax.axis_index("s")

        def work(e_h, o_h):
            R0 = tuple(jnp.zeros((16,), _F32) for _ in range(dw))

            @pl.loop(0, nch, init_carry=R0)
            def _chunk(ci, R):
                base = s * EPT + ci * CH
                pltpu.sync_copy(sep_h.at[pl.ds(base, CH)], sei)
                pltpu.sync_copy(kp_h.at[pl.ds(base, CH)], kbuf)
                pltpu.async_copy(e_h.at[sei], ebuf, s1).wait()

                @pl.loop(0, CH // 16, init_carry=R)
                def _grp(g, Rg):
                    kv16 = kbuf[pl.ds(g * 16, 16)]
                    Rg = list(Rg)
                    for l in range(16):
                        r = g * 16 + l
                        kb = _bcast(kv16, l)
                        for j in range(dw):
                            sl = pl.ds(j * 16, 16)
                            Rg[j] = Rg[j] * kb + ebuf[r, sl]
                            ebuf[r, sl] = Rg[j]
                    return tuple(Rg)

                pltpu.sync_copy(ebuf, o_h.at[pl.ds(base, CH)])
                return _grp

        @pl.when(c == 0)
        def _c0():
            work(e0_h, o0_h)

        @pl.when(c == 1)
        def _c1():
            work(e1_h, o1_h)

    return run(e0, e1, sep, keep)


def _sc_pick(o0, o1, ell, *, dh, Ep, NFp):
    NPT = NFp // _NS
    nch = NPT // 8
    dw = dh // 16
    mesh = plsc.VectorSubcoreMesh(core_axis_name="c", subcore_axis_name="s")
    out_type = jax.ShapeDtypeStruct((2 * NFp, dh), _F32)
    scratch = [
        pltpu.VMEM((128, dh), _F32),
        pltpu.VMEM((8, dh), _F32),
        pltpu.VMEM((128,), _I32),
        pltpu.SemaphoreType.DMA,
    ]

    @functools.partial(pl.kernel, out_type=out_type, mesh=mesh,
                       scratch_types=scratch)
    def run(o0_h, o1_h, ell_h, agg_h, gbuf, abuf, gi, s1):
        c = lax.axis_index("c")
        s = lax.axis_index("s")

        def work(o_h):
            @pl.loop(0, nch)
            def _chunk(ci):
                n0 = s * NPT + ci * 8
                pltpu.sync_copy(ell_h.at[pl.ds(n0 * 16, 128)], gi)
                pltpu.async_copy(o_h.at[gi], gbuf, s1).wait()

                @pl.loop(0, 8)
                def _node(k):
                    for j in range(dw):
                        sl = pl.ds(j * 16, 16)
                        t = gbuf[k * 16, sl]
                        for m in range(1, 16):
                            t = t + gbuf[k * 16 + m, sl]
                        abuf[k, sl] = t

                pltpu.sync_copy(abuf, agg_h.at[pl.ds(c * NFp + n0, 8)])

        @pl.when(c == 0)
        def _c0():
            work(o0_h)

        @pl.when(c == 1)
        def _c1():
            work(o1_h)

    return run(o0, o1, ell)


def _sc_seg(e0, e1, spkg, *, dh, Ep, NFp):
    sep, keep, ell = spkg
    o0, o1 = _sc_scan(e0, e1, sep, keep, dh=dh, Ep=Ep, CH=128)
    return _sc_pick(o0, o1, ell, dh=dh, Ep=Ep, NFp=NFp)


def _sc_node(T, g, D, S, *, dout, NFp, CH=32):
    RPW = NFp // _NW
    nch = RPW // CH
    dw = dout // 16
    with_s = S is not None
    mesh = plsc.VectorSubcoreMesh(core_axis_name="c", subcore_axis_name="s")
    scratch = [pltpu.VMEM((CH, dout), _F32), pltpu.VMEM((CH, dout), _F32),
               pltpu.VMEM((CH, dout), _F32), pltpu.VMEM((CH,), _I32),
               pltpu.SemaphoreType.DMA]
    out_type = jax.ShapeDtypeStruct((NFp, dout), _F32)

    @functools.partial(pl.kernel, out_type=out_type, mesh=mesh,
                       scratch_types=scratch)
    def run(*refs):
        if with_s:
            (T_h, g_h, D_h, S_h, o_h, tbuf, dbuf, sbuf, gi, s1) = refs
        else:
            (T_h, g_h, D_h, o_h, tbuf, dbuf, sbuf, gi, s1) = refs
            S_h = None
        c = lax.axis_index("c")
        s = lax.axis_index("s")
        wid = s * _NC + c

        @pl.loop(0, nch)
        def _chunk(ci):
            base = wid * RPW + ci * CH
            pltpu.sync_copy(g_h.at[pl.ds(base, CH)], gi)
            h = pltpu.async_copy(T_h.at[gi], tbuf, s1)
            pltpu.sync_copy(D_h.at[pl.ds(base, CH)], dbuf)
            if with_s:
                pltpu.sync_copy(S_h.at[pl.ds(base, CH)], sbuf)
            h.wait()

            @pl.loop(0, CH)
            def _row(r):
                for j in range(dw):
                    sl = pl.ds(j * 16, 16)
                    v = tbuf[r, sl] + dbuf[r, sl]
                    v = jnp.maximum(v, v * _LEAK)
                    if with_s:
                        v = v + sbuf[r, sl]
                        v = jnp.maximum(v, v * _LEAK)
                    tbuf[r, sl] = v

            pltpu.sync_copy(tbuf, o_h.at[pl.ds(base, CH)])

    if with_s:
        return run(T, g, D, S)
    return run(T, g, D)


def _sort_pkg(dst, Er, Ep, NFp):
    EPT = Ep // _NS
    perm = jnp.argsort(dst).astype(_I32)
    sdst_r = dst[perm].astype(_I32)
    sep = _pad1(perm, Ep, Er)
    sdst_p = _pad1(sdst_r, Ep, NFp)
    prev = jnp.concatenate([jnp.full((1,), -1, _I32), sdst_p[:-1]])
    r = jnp.arange(Ep, dtype=_I32)
    newseg = (sdst_p != prev) | (r % EPT == 0)
    keep = jnp.where(newseg, 0.0, 1.0).astype(_F32)
    ar = jnp.arange(NFp, dtype=_I32)
    starts = jnp.searchsorted(sdst_r, ar).astype(_I32)
    ends = jnp.searchsorted(sdst_r, ar + 1).astype(_I32)
    pick_end = jnp.where(ends > starts, ends - 1, Er)
    B = jnp.arange(1, _NS, dtype=_I32) * EPT
    cond = (starts[:, None] < B[None, :]) & (B[None, :] < ends[:, None])
    pick_b = jnp.where(cond, B[None, :] - 1, Er)
    ell = jnp.concatenate([pick_end[:, None], pick_b], axis=1)
    return sep, keep, ell.reshape(-1)


def _plain_mpl(x, e_list, src, dst, spkg, mp, Nr, Np, Er, Ep, CH):
    din = mp['W_e'].shape[0] // 3
    dout = mp['W_e'].shape[1]
    dh = dout // 2
    dhp = max(dh, 128)
    W1, W2, W3 = mp['W_e'][:din], mp['W_e'][din:2 * din], mp['W_e'][2 * din:]
    AB = _tc_mm([(x, jnp.concatenate(
        [_wpadc(W1[:, :dh], dhp), _wpadc(W1[:, dh:], dhp),
         _wpadc(W2[:, :dh], dhp), _wpadc(W2[:, dh:], dhp)], axis=1))],
        jnp.zeros((4 * dhp,), _F32), act=False, m_real=Nr)
    A0, A1 = AB[:, :dhp], AB[:, dhp:2 * dhp]
    B0, B1 = AB[:, 2 * dhp:3 * dhp], AB[:, 3 * dhp:]
    C0 = _tc_mm(_eparts(e_list, jnp.pad(W3[:, :dh], ((0, 0), (0, dhp - dh)))),
                jnp.zeros((dhp,), _F32), act=False, m_real=Er)
    C1 = _tc_mm(_eparts(e_list, jnp.pad(W3[:, dh:], ((0, 0), (0, dhp - dh)))),
                jnp.zeros((dhp,), _F32), act=False, m_real=Er)
    srcg = _pad1(src, Ep, Nr)
    dstg = _pad1(dst, Ep, Nr)
    valid = _pad1(jnp.ones((Er,), _I32), Ep, 0).astype(_F32)
    be0 = jnp.pad(mp['b_e'][:dh], (0, dhp - dh))
    be1 = jnp.pad(mp['b_e'][dh:], (0, dhp - dh))
    e0, e1 = _sc_edge(A0, A1, B0, B1, C0, C1, srcg, dstg, dstg,
                      be0, be1, valid, dh=dhp, Ep=Ep, CH=CH, gather_c=False)
    agg = _sc_seg(e0, e1, spkg, dh=dhp, Ep=Ep, NFp=Np)
    Wn = mp['W_n']
    Wb0 = jnp.pad(Wn[din:din + dh], ((0, dhp - dh), (0, 0)))
    Wb1 = jnp.pad(Wn[din + dh:], ((0, dhp - dh), (0, 0)))
    x_new = _tc_mm([(x, Wn[:din]), (agg[:Np], Wb0), (agg[Np:], Wb1)],
                   mp['b_n'], act=True, m_real=Nr)
    return x_new, [(e0, dh), (e1, dh)]


def _gather_mpl(xc, ec_list, gn, ge, src_f, dst_f, spkg, mp, Ncr, Nfr, Nfp,
                Ecr, Efr, Efp, CH, S=None):
    din = mp['W_e'].shape[0] // 3
    dout = mp['W_e'].shape[1]
    dh = dout // 2
    dhp = max(dh, 128)
    W1, W2, W3 = mp['W_e'][:din], mp['W_e'][din:2 * din], mp['W_e'][2 * din:]
    ABT = _tc_mm([(xc, jnp.concatenate(
        [_wpadc(W1[:, :dh], dhp), _wpadc(W1[:, dh:], dhp),
         _wpadc(W2[:, :dh], dhp), _wpadc(W2[:, dh:], dhp),
         mp['W_n'][:din]], axis=1))],
        jnp.zeros((4 * dhp + dout,), _F32), act=False, m_real=Ncr)
    A0, A1 = ABT[:, :dhp], ABT[:, dhp:2 * dhp]
    B0, B1 = ABT[:, 2 * dhp:3 * dhp], ABT[:, 3 * dhp:4 * dhp]
    T = ABT[:, 4 * dhp:]
    C0 = _tc_mm(_eparts(ec_list, jnp.pad(W3[:, :dh], ((0, 0), (0, dhp - dh)))),
                jnp.zeros((dhp,), _F32), act=False, m_real=Ecr)
    C1 = _tc_mm(_eparts(ec_list, jnp.pad(W3[:, dh:], ((0, 0), (0, dhp - dh)))),
                jnp.zeros((dhp,), _F32), act=False, m_real=Ecr)
    srcg = _pad1(gn[src_f], Efp, Ncr)
    dstg = _pad1(gn[dst_f], Efp, Ncr)
    egi = _pad1(ge, Efp, Ecr)
    valid = _pad1(jnp.ones((Efr,), _I32), Efp, 0).astype(_F32)
    be0 = jnp.pad(mp['b_e'][:dh], (0, dhp - dh))
    be1 = jnp.pad(mp['b_e'][dh:], (0, dhp - dh))
    e0, e1 = _sc_edge(A0, A1, B0, B1, C0, C1, srcg, dstg, egi,
                      be0, be1, valid, dh=dhp, Ep=Efp, CH=CH, gather_c=True)
    agg = _sc_seg(e0, e1, spkg, dh=dhp, Ep=Efp, NFp=Nfp)
    Wnb = mp['W_n'][din:]
    Wb0 = jnp.pad(Wnb[:dh], ((0, dhp - dh), (0, 0)))
    Wb1 = jnp.pad(Wnb[dh:], ((0, dhp - dh), (0, 0)))
    D = _tc_mm([(agg[:Nfp], Wb0), (agg[Nfp:], Wb1)], mp['b_n'],
               act=False, m_real=Nfr)
    gnode = _pad1(gn, Nfp, Ncr)
    x_new = _sc_node(T, gnode, D, S, dout=dout, NFp=Nfp)
    return x_new, [(e0, dh), (e1, dh)]


def kernel(z, edge_attr, params, edge_index2, edge_index1, edge_index0,
           m_id1, m_id0, e_idx1, e_idx0):
    p = params
    N2r, N1r, N0r = 2500, 5000, 10000
    E2r, E1r, E0r = 40000, 80000, 160000
    N2p, N1p, N0p = 2560, 5120, 10240
    E2p, E1p, E0p = 40960, 81920, 163840

    ei2 = edge_index2.astype(_I32)
    ei1 = edge_index1.astype(_I32)
    ei0 = edge_index0.astype(_I32)

    gn1 = jnp.full((N1r,), N2r, _I32).at[m_id1].set(jnp.arange(N2r, dtype=_I32))
    gn0 = jnp.full((N0r,), N1r, _I32).at[m_id0].set(jnp.arange(N1r, dtype=_I32))
    ge1 = jnp.full((E1r,), E2r, _I32).at[e_idx1].set(jnp.arange(E2r, dtype=_I32))
    ge0 = jnp.full((E0r,), E1r, _I32).at[e_idx0].set(jnp.arange(E1r, dtype=_I32))

    s2 = _sort_pkg(ei2[1], E2r, E2p, N2p)
    s1 = _sort_pkg(ei1[1], E1r, E1p, N1p)
    s0 = _sort_pkg(ei0[1], E0r, E0p, N0p)

    z2 = jnp.pad(z.reshape(-1, 1), ((0, 0), (0, 7)))
    W1z = jnp.pad(p['up_W1'], ((0, 7), (0, 0)))
    h = _tc_mm([(z2, W1z)], p['up_b1'], act=True, m_real=None, bm=128)
    haug = jnp.concatenate([h.T, jnp.ones((1, h.shape[0]), _F32)], axis=0)
    haug = jnp.pad(haug, ((0, 7), (0, 0)))
    W2a = jnp.concatenate([p['up_W2'].T, p['up_b2'][:, None]], axis=1)
    W2a = jnp.pad(W2a, ((0, N2p - N2r), (0, 7)))
    x = _tc_mm([(W2a, haug)], jnp.zeros((haug.shape[1],), _F32), act=False,
               m_real=N2r)

    e_list = [(_pad_rows(edge_attr, E2p), edge_attr.shape[1])]

    x, e_list = _plain_mpl(x, e_list, ei2[0], ei2[1], s2, p['bottom'], N2r,
                           N2p, E2r, E2p, CH=64)

    rp = p['r0']
    x_skip, _ = _gather_mpl(x, e_list, gn1, ge1, ei1[0], ei1[1], s1,
                            rp['skip'], N2r, N1r, N1p, E2r, E1r, E1p, CH=128)
    x1, e1_list = _plain_mpl(x, e_list, ei2[0], ei2[1], s2, rp['mpl1'], N2r,
                             N2p, E2r, E2p, CH=128)
    x, e_list = _gather_mpl(x1, e1_list, gn1, ge1, ei1[0], ei1[1], s1,
                            rp['mpl2'], N2r, N1r, N1p, E2r, E1r, E1p, CH=128,
                            S=x_skip)

    rp = p['r1']
    x_skip, _ = _gather_mpl(x, e_list, gn0, ge0, ei0[0], ei0[1], s0,
                            rp['skip'], N1r, N0r, N0p, E1r, E0r, E0p, CH=128)
    x1, e1_list = _plain_mpl(x, e_list, ei1[0], ei1[1], s1, rp['mpl1'], N1r,
                             N1p, E1r, E1p, CH=128)
    x, e_list = _gather_mpl(x1, e1_list, gn0, ge0, ei0[0], ei0[1], s0,
                            rp['mpl2'], N1r, N0r, N0p, E1r, E0r, E0p, CH=128,
                            S=x_skip)

    x, e_list = _plain_mpl(x, e_list, ei0[0], ei0[1], s0, p['final'], N0r,
                           N0p, E0r, E0p, CH=128)

    xn = _tc_head([(x, x.shape[1])], p['nd_W1'], p['nd_b1'], p['nd_W2'], p['nd_b2'],
                  p['nd_ln_g'], p['nd_ln_b'])
    en = _tc_head(e_list, p['ed_W1'], p['ed_b1'], p['ed_W2'], p['ed_b2'],
                  p['ed_ln_g'], p['ed_ln_b'])
    return xn[:N0r], en[:E0r]


# speedup vs baseline: 1.0328x; 1.0328x over previous
import functools

import jax
import jax.numpy as jnp
from jax import lax
from jax.experimental import pallas as pl
from jax.experimental.pallas import tpu as pltpu
from jax.experimental.pallas import tpu_sc as plsc

_LEAK = 0.01
_F32 = jnp.float32
_I32 = jnp.int32
_NC, _NS, _NW = 2, 16, 32


def _lrelu(x):
    return jnp.maximum(x, _LEAK * x)


def _pad_rows(a, n):
    return jnp.pad(a, ((0, n - a.shape[0]), (0, 0)))


def _pad1(a, n, val):
    a = a.astype(_I32)
    return jnp.concatenate([a, jnp.full((n - a.shape[0],), val, _I32)])


def _eparts(e_list, W):
    parts, off = [], 0
    for ep, kreal in e_list:
        wpart = W[off:off + kreal]
        wpart = jnp.pad(wpart, ((0, ep.shape[1] - kreal), (0, 0)))
        parts.append((ep, wpart))
        off += kreal
    return parts


def _wpadc(W, w):
    return jnp.pad(W, ((0, 0), (0, w - W.shape[1])))


def _tc_mm(parts, b, act, m_real, bm=256):
    M = parts[0][0].shape[0]
    N = parts[0][1].shape[1]
    nP = len(parts)

    def kern(*refs):
        o_ref = refs[-1]
        acc = jnp.zeros((bm, N), _F32) + refs[2 * nP][...]
        for i in range(nP):
            acc = acc + jnp.dot(refs[2 * i][...], refs[2 * i + 1][...],
                                preferred_element_type=_F32)
        if act:
            acc = _lrelu(acc)
        if m_real is not None:
            rows = pl.program_id(0) * bm + lax.broadcasted_iota(_I32, (bm, N), 0)
            acc = jnp.where(rows < m_real, acc, 0.0)
        o_ref[...] = acc

    in_specs = []
    args = []
    for X, W in parts:
        K = X.shape[1]
        in_specs.append(pl.BlockSpec((bm, K), lambda i: (i, 0)))
        in_specs.append(pl.BlockSpec((K, N), lambda i: (0, 0)))
        args += [X, W]
    in_specs.append(pl.BlockSpec((1, N), lambda i: (0, 0)))
    args.append(b.reshape(1, N))
    return pl.pallas_call(
        kern, grid=(M // bm,), in_specs=in_specs,
        out_specs=pl.BlockSpec((bm, N), lambda i: (i, 0)),
        out_shape=jax.ShapeDtypeStruct((M, N), _F32))(*args)


def _tc_head(xparts, W1, b1, W2, b2, g, bb, bm=256):
    M = xparts[0][0].shape[0]
    H = W1.shape[1]
    NO = W2.shape[1]
    parts = _eparts(xparts, W1)
    nP = len(parts)

    def kern(*refs):
        o_ref = refs[-1]
        t = jnp.zeros((bm, H), _F32) + refs[2 * nP][...]
        for i in range(nP):
            t = t + jnp.dot(refs[2 * i][...], refs[2 * i + 1][...],
                            preferred_element_type=_F32)
        t = _lrelu(t)
        u = jnp.dot(t, refs[2 * nP + 1][...],
                    preferred_element_type=_F32) + refs[2 * nP + 2][...]
        mu = jnp.mean(u, axis=-1, keepdims=True)
        var = jnp.mean((u - mu) ** 2, axis=-1, keepdims=True)
        o_ref[...] = ((u - mu) * lax.rsqrt(var + 1e-5) * refs[2 * nP + 3][...]
                      + refs[2 * nP + 4][...])

    in_specs = []
    args = []
    for X, W in parts:
        K = X.shape[1]
        in_specs.append(pl.BlockSpec((bm, K), lambda i: (i, 0)))
        in_specs.append(pl.BlockSpec((K, H), lambda i: (0, 0)))
        args += [X, W]
    in_specs += [pl.BlockSpec((1, H), lambda i: (0, 0)),
                 pl.BlockSpec((H, NO), lambda i: (0, 0)),
                 pl.BlockSpec((1, NO), lambda i: (0, 0)),
                 pl.BlockSpec((1, NO), lambda i: (0, 0)),
                 pl.BlockSpec((1, NO), lambda i: (0, 0))]
    args += [b1.reshape(1, H), W2, b2.reshape(1, NO), g.reshape(1, NO),
             bb.reshape(1, NO)]
    return pl.pallas_call(
        kern, grid=(M // bm,), in_specs=in_specs,
        out_specs=pl.BlockSpec((bm, NO), lambda i: (i, 0)),
        out_shape=jax.ShapeDtypeStruct((M, NO), _F32))(*args)


def _bcast(v16, l):
    return v16.at[jnp.full((16,), l, _I32)].get(mode='promise_in_bounds')


def _sc_edge(A0, A1, B0, B1, C0, C1, pk, be0, be1, *, dh, Ep, CH, gather_c):
    EPT = Ep // _NS
    nch = EPT // CH
    dw = dh // 16
    mesh = plsc.VectorSubcoreMesh(core_axis_name="c", subcore_axis_name="s")
    out_type = (jax.ShapeDtypeStruct((Ep, dh), _F32),
                jax.ShapeDtypeStruct((Ep, dh), _F32))
    bufset = [pltpu.VMEM((CH, dh), _F32), pltpu.VMEM((CH, dh), _F32),
              pltpu.VMEM((CH, dh), _F32), pltpu.VMEM((4, CH), _I32),
              pltpu.SemaphoreType.DMA, pltpu.SemaphoreType.DMA,
              pltpu.SemaphoreType.DMA]
    scratch = bufset + bufset + [pltpu.VMEM((dh,), _F32)]

    @functools.partial(pl.kernel, out_type=out_type, mesh=mesh,
                       scratch_types=scratch)
    def run(A0_h, A1_h, B0_h, B1_h, C0_h, C1_h, pk_h, be0_h, be1_h,
            e0_h, e1_h,
            aA, bA, cA, iA, xA1, xA2, xA3,
            aB, bB, cB, iB, xB1, xB2, xB3, biasv):
        c = lax.axis_index("c")
        s = lax.axis_index("s")
        setA = (aA, bA, cA, iA, xA1, xA2, xA3)
        setB = (aB, bB, cB, iB, xB1, xB2, xB3)

        def work(A_h, B_h, C_h, be_h, enew_h):
            pltpu.sync_copy(be_h, biasv)

            def issue(ki, st):
                ab, bb, cb, ib, x1, x2, x3 = st
                kg = s * nch + ki
                pltpu.sync_copy(pk_h.at[kg], ib)
                h1 = pltpu.async_copy(A_h.at[ib.at[0]], ab, x1)
                h2 = pltpu.async_copy(B_h.at[ib.at[1]], bb, x2)
                if gather_c:
                    h3 = pltpu.async_copy(C_h.at[ib.at[2]], cb, x3)
                else:
                    h3 = pltpu.async_copy(C_h.at[pl.ds(kg * CH, CH)], cb, x3)
                return h1, h2, h3

            def wait_rebuild(st):
                ab, bb, cb, ib, x1, x2, x3 = st
                pltpu.make_async_copy(A_h.at[ib.at[0]], ab, x1).wait()
                pltpu.make_async_copy(B_h.at[ib.at[1]], bb, x2).wait()
                if gather_c:
                    pltpu.make_async_copy(C_h.at[ib.at[2]], cb, x3).wait()
                else:
                    pltpu.make_async_copy(C_h.at[pl.ds(0, CH)], cb, x3).wait()

            def compute(ki, st):
                ab, bb, cb, ib, x1, x2, x3 = st
                kg = s * nch + ki

                @pl.loop(0, CH // 16)
                def _grp(g):
                    vv16 = ib[3, pl.ds(g * 16, 16)].astype(_F32)

                    @pl.loop(0, 16)
                    def _row(l):
                        r = g * 16 + l
                        vb = _bcast(vv16, l)
                        for j in range(dw):
                            sl = pl.ds(j * 16, 16)
                            v = (ab[r, sl] + bb[r, sl] + cb[r, sl]
                                 + biasv[sl])
                            ab[r, sl] = jnp.maximum(v, v * _LEAK) * vb

                pltpu.sync_copy(ab, enew_h.at[pl.ds(kg * CH, CH)])

            issue(0, setA)

            @pl.loop(0, nch // 2)
            def _body(cj):
                hB = issue(2 * cj + 1, setB)
                wait_rebuild(setA)
                compute(2 * cj, setA)

                @pl.when(2 * cj + 2 < nch)
                def _():
                    issue(2 * cj + 2, setA)

                for h in hB:
                    h.wait()
                compute(2 * cj + 1, setB)

        @pl.when(c == 0)
        def _c0():
            work(A0_h, B0_h, C0_h, be0_h, e0_h)

        @pl.when(c == 1)
        def _c1():
            work(A1_h, B1_h, C1_h, be1_h, e1_h)

    return run(A0, A1, B0, B1, C0, C1, pk, be0, be1)


def _sc_scan(e0, e1, pk2, *, dh, Ep, CH):
    EPT = Ep // _NS
    nch = EPT // CH
    dw = dh // 16
    mesh = plsc.VectorSubcoreMesh(core_axis_name="c", subcore_axis_name="s")
    out_type = (jax.ShapeDtypeStruct((Ep, dh), _F32),
                jax.ShapeDtypeStruct((Ep, dh), _F32))
    bufset = [pltpu.VMEM((CH, dh), _F32), pltpu.VMEM((2, CH), _I32),
              pltpu.SemaphoreType.DMA]
    scratch = bufset + bufset

    @functools.partial(pl.kernel, out_type=out_type, mesh=mesh,
                       scratch_types=scratch)
    def run(e0_h, e1_h, pk_h, o0_h, o1_h, eA, iA, xA, eB, iB, xB):
        c = lax.axis_index("c")
        s = lax.axis_index("s")
        setA = (eA, iA, xA)
        setB = (eB, iB, xB)

        def work(e_h, o_h):
            def issue(ki, st):
                eb, ib, x1 = st
                pltpu.sync_copy(pk_h.at[s * nch + ki], ib)
                return pltpu.async_copy(e_h.at[ib.at[0]], eb, x1)

            def wait_rebuild(st):
                eb, ib, x1 = st
                pltpu.make_async_copy(e_h.at[ib.at[0]], eb, x1).wait()

            def compute(ki, st, R):
                eb, ib, x1 = st

                @pl.loop(0, CH // 16, init_carry=R)
                def _grp(g, Rg):
                    kv16 = lax.bitcast_convert_type(ib[1, pl.ds(g * 16, 16)],
                                                    _F32)

                    @pl.loop(0, 16, init_carry=Rg)
                    def _row(l, Rr):
                        r = g * 16 + l
                        kb = _bcast(kv16, l)
                        Rr = list(Rr)
                        for j in range(dw):
                            sl = pl.ds(j * 16, 16)
                            Rr[j] = Rr[j] * kb + eb[r, sl]
                            eb[r, sl] = Rr[j]
                        return tuple(Rr)

                    return _row

                pltpu.sync_copy(eb, o_h.at[pl.ds((s * nch + ki) * CH, CH)])
                return _grp

            issue(0, setA)
            R0 = tuple(jnp.zeros((16,), _F32) for _ in range(dw))

            @pl.loop(0, nch // 2, init_carry=R0)
            def _body(cj, R):
                hB = issue(2 * cj + 1, setB)
                wait_rebuild(setA)
                R = compute(2 * cj, setA, R)

                @pl.when(2 * cj + 2 < nch)
                def _():
                    issue(2 * cj + 2, setA)

                hB.wait()
                return compute(2 * cj + 1, setB, R)

        @pl.when(c == 0)
        def _c0():
            work(e0_h, o0_h)

        @pl.when(c == 1)
        def _c1():
            work(e1_h, o1_h)

    return run(e0, e1, pk2)


def _sc_pick(o0, o1, ell, *, dh, Ep, NFp):
    NPT = NFp // _NS
    nch = NPT // 8
    dw = dh // 16
    mesh = plsc.VectorSubcoreMesh(core_axis_name="c", subcore_axis_name="s")
    out_type = jax.ShapeDtypeStruct((2 * NFp, dh), _F32)
    bufset = [pltpu.VMEM((128, dh), _F32), pltpu.VMEM((128,), _I32),
              pltpu.SemaphoreType.DMA]
    scratch = bufset + bufset + [pltpu.VMEM((8, dh), _F32)]

    @functools.partial(pl.kernel, out_type=out_type, mesh=mesh,
                       scratch_types=scratch)
    def run(o0_h, o1_h, ell_h, agg_h, gA, iA, xA, gB, iB, xB, abuf):
        c = lax.axis_index("c")
        s = lax.axis_index("s")
        setA = (gA, iA, xA)
        setB = (gB, iB, xB)

        def work(o_h):
            def issue(ki, st):
                gb, ib, x1 = st
                n0 = s * NPT + ki * 8
                pltpu.sync_copy(ell_h.at[pl.ds(n0 * 16, 128)], ib)
                return pltpu.async_copy(o_h.at[ib], gb, x1)

            def wait_rebuild(st):
                gb, ib, x1 = st
                pltpu.make_async_copy(o_h.at[ib], gb, x1).wait()

            def compute(ki, st):
                gb, ib, x1 = st
                n0 = s * NPT + ki * 8

                @pl.loop(0, 8)
                def _node(k):
                    for j in range(dw):
                        sl = pl.ds(j * 16, 16)
                        t = gb[k * 16, sl]
                        for m in range(1, 8):
                            t = t + gb[k * 16 + m, sl]
                        t2 = gb[k * 16 + 8, sl]
                        for m in range(9, 16):
                            t2 = t2 + gb[k * 16 + m, sl]
                        abuf[k, sl] = t + t2

                pltpu.sync_copy(abuf, agg_h.at[pl.ds(c * NFp + n0, 8)])

            issue(0, setA)

            @pl.loop(0, nch // 2)
            def _body(cj):
                hB = issue(2 * cj + 1, setB)
                wait_rebuild(setA)
                compute(2 * cj, setA)

                @pl.when(2 * cj + 2 < nch)
                def _():
                    issue(2 * cj + 2, setA)

                hB.wait()
                compute(2 * cj + 1, setB)

        @pl.when(c == 0)
        def _c0():
            work(o0_h)

        @pl.when(c == 1)
        def _c1():
            work(o1_h)

    return run(o0, o1, ell)


def _sc_seg(e0, e1, spkg, *, dh, Ep, NFp):
    pk2, ell = spkg
    o0, o1 = _sc_scan(e0, e1, pk2, dh=dh, Ep=Ep, CH=128)
    return _sc_pick(o0, o1, ell, dh=dh, Ep=Ep, NFp=NFp)


def _pack4(srcg, dstg, eg, valid_i, CH):
    n = srcg.shape[0] // CH
    return jnp.stack([srcg.reshape(n, CH), dstg.reshape(n, CH),
                      eg.reshape(n, CH), valid_i.reshape(n, CH)], axis=1)


def _sc_node(T, g, D, S, *, dout, NFp, CH=32):
    RPW = NFp // _NW
    nch = RPW // CH
    dw = dout // 16
    with_s = S is not None
    mesh = plsc.VectorSubcoreMesh(core_axis_name="c", subcore_axis_name="s")
    scratch = [pltpu.VMEM((CH, dout), _F32), pltpu.VMEM((CH, dout), _F32),
               pltpu.VMEM((CH, dout), _F32), pltpu.VMEM((CH,), _I32),
               pltpu.SemaphoreType.DMA]
    out_type = jax.ShapeDtypeStruct((NFp, dout), _F32)

    @functools.partial(pl.kernel, out_type=out_type, mesh=mesh,
                       scratch_types=scratch)
    def run(*refs):
        if with_s:
            (T_h, g_h, D_h, S_h, o_h, tbuf, dbuf, sbuf, gi, s1) = refs
        else:
            (T_h, g_h, D_h, o_h, tbuf, dbuf, sbuf, gi, s1) = refs
            S_h = None
        c = lax.axis_index("c")
        s = lax.axis_index("s")
        wid = s * _NC + c

        @pl.loop(0, nch)
        def _chunk(ci):
            base = wid * RPW + ci * CH
            pltpu.sync_copy(g_h.at[pl.ds(base, CH)], gi)
            h = pltpu.async_copy(T_h.at[gi], tbuf, s1)
            pltpu.sync_copy(D_h.at[pl.ds(base, CH)], dbuf)
            if with_s:
                pltpu.sync_copy(S_h.at[pl.ds(base, CH)], sbuf)
            h.wait()

            @pl.loop(0, CH)
            def _row(r):
                for j in range(dw):
                    sl = pl.ds(j * 16, 16)
                    v = tbuf[r, sl] + dbuf[r, sl]
                    v = jnp.maximum(v, v * _LEAK)
                    if with_s:
                        v = v + sbuf[r, sl]
                        v = jnp.maximum(v, v * _LEAK)
                    tbuf[r, sl] = v

            pltpu.sync_copy(tbuf, o_h.at[pl.ds(base, CH)])

    if with_s:
        return run(T, g, D, S)
    return run(T, g, D)


def _sort_pkg(dst, Er, Ep, NFp):
    EPT = Ep // _NS
    perm = jnp.argsort(dst).astype(_I32)
    sdst_r = dst[perm].astype(_I32)
    sep = _pad1(perm, Ep, Er)
    sdst_p = _pad1(sdst_r, Ep, NFp)
    prev = jnp.concatenate([jnp.full((1,), -1, _I32), sdst_p[:-1]])
    r = jnp.arange(Ep, dtype=_I32)
    newseg = (sdst_p != prev) | (r % EPT == 0)
    keep = jnp.where(newseg, 0.0, 1.0).astype(_F32)
    ar = jnp.arange(NFp, dtype=_I32)
    starts = jnp.searchsorted(sdst_r, ar).astype(_I32)
    ends = jnp.searchsorted(sdst_r, ar + 1).astype(_I32)
    pick_end = jnp.where(ends > starts, ends - 1, Er)
    B = jnp.arange(1, _NS, dtype=_I32) * EPT
    cond = (starts[:, None] < B[None, :]) & (B[None, :] < ends[:, None])
    pick_b = jnp.where(cond, B[None, :] - 1, Er)
    ell = jnp.concatenate([pick_end[:, None], pick_b], axis=1)
    kbits = lax.bitcast_convert_type(keep, _I32)
    n = Ep // 128
    pk2 = jnp.stack([sep.reshape(n, 128), kbits.reshape(n, 128)], axis=1)
    return pk2, ell.reshape(-1)


def _plain_mpl(x, e_list, src, dst, spkg, mp, Nr, Np, Er, Ep, CH):
    din = mp['W_e'].shape[0] // 3
    dout = mp['W_e'].shape[1]
    dh = dout // 2
    dhp = max(dh, 128)
    W1, W2, W3 = mp['W_e'][:din], mp['W_e'][din:2 * din], mp['W_e'][2 * din:]
    AB = _tc_mm([(x, jnp.concatenate(
        [_wpadc(W1[:, :dh], dhp), _wpadc(W1[:, dh:], dhp),
         _wpadc(W2[:, :dh], dhp), _wpadc(W2[:, dh:], dhp)], axis=1))],
        jnp.zeros((4 * dhp,), _F32), act=False, m_real=Nr)
    A0, A1 = AB[:, :dhp], AB[:, dhp:2 * dhp]
    B0, B1 = AB[:, 2 * dhp:3 * dhp], AB[:, 3 * dhp:]
    C0 = _tc_mm(_eparts(e_list, jnp.pad(W3[:, :dh], ((0, 0), (0, dhp - dh)))),
                jnp.zeros((dhp,), _F32), act=False, m_real=Er)
    C1 = _tc_mm(_eparts(e_list, jnp.pad(W3[:, dh:], ((0, 0), (0, dhp - dh)))),
                jnp.zeros((dhp,), _F32), act=False, m_real=Er)
    srcg = _pad1(src, Ep, Nr)
    dstg = _pad1(dst, Ep, Nr)
    valid_i = _pad1(jnp.ones((Er,), _I32), Ep, 0)
    pk = _pack4(srcg, dstg, dstg, valid_i, CH)
    be0 = jnp.pad(mp['b_e'][:dh], (0, dhp - dh))
    be1 = jnp.pad(mp['b_e'][dh:], (0, dhp - dh))
    e0, e1 = _sc_edge(A0, A1, B0, B1, C0, C1, pk,
                      be0, be1, dh=dhp, Ep=Ep, CH=CH, gather_c=False)
    agg = _sc_seg(e0, e1, spkg, dh=dhp, Ep=Ep, NFp=Np)
    Wn = mp['W_n']
    Wb0 = jnp.pad(Wn[din:din + dh], ((0, dhp - dh), (0, 0)))
    Wb1 = jnp.pad(Wn[din + dh:], ((0, dhp - dh), (0, 0)))
    x_new = _tc_mm([(x, Wn[:din]), (agg[:Np], Wb0), (agg[Np:], Wb1)],
                   mp['b_n'], act=True, m_real=Nr)
    return x_new, [(e0, dh), (e1, dh)]


def _gather_mpl(xc, ec_list, gn, ge, src_f, dst_f, spkg, mp, Ncr, Nfr, Nfp,
                Ecr, Efr, Efp, CH, S=None):
    din = mp['W_e'].shape[0] // 3
    dout = mp['W_e'].shape[1]
    dh = dout // 2
    dhp = max(dh, 128)
    W1, W2, W3 = mp['W_e'][:din], mp['W_e'][din:2 * din], mp['W_e'][2 * din:]
    ABT = _tc_mm([(xc, jnp.concatenate(
        [_wpadc(W1[:, :dh], dhp), _wpadc(W1[:, dh:], dhp),
         _wpadc(W2[:, :dh], dhp), _wpadc(W2[:, dh:], dhp),
         mp['W_n'][:din]], axis=1))],
        jnp.zeros((4 * dhp + dout,), _F32), act=False, m_real=Ncr)
    A0, A1 = ABT[:, :dhp], ABT[:, dhp:2 * dhp]
    B0, B1 = ABT[:, 2 * dhp:3 * dhp], ABT[:, 3 * dhp:4 * dhp]
    T = ABT[:, 4 * dhp:]
    C0 = _tc_mm(_eparts(ec_list, jnp.pad(W3[:, :dh], ((0, 0), (0, dhp - dh)))),
                jnp.zeros((dhp,), _F32), act=False, m_real=Ecr)
    C1 = _tc_mm(_eparts(ec_list, jnp.pad(W3[:, dh:], ((0, 0), (0, dhp - dh)))),
                jnp.zeros((dhp,), _F32), act=False, m_real=Ecr)
    srcg = _pad1(gn[src_f], Efp, Ncr)
    dstg = _pad1(gn[dst_f], Efp, Ncr)
    egi = _pad1(ge, Efp, Ecr)
    valid_i = _pad1(jnp.ones((Efr,), _I32), Efp, 0)
    pk = _pack4(srcg, dstg, egi, valid_i, CH)
    be0 = jnp.pad(mp['b_e'][:dh], (0, dhp - dh))
    be1 = jnp.pad(mp['b_e'][dh:], (0, dhp - dh))
    e0, e1 = _sc_edge(A0, A1, B0, B1, C0, C1, pk,
                      be0, be1, dh=dhp, Ep=Efp, CH=CH, gather_c=True)
    agg = _sc_seg(e0, e1, spkg, dh=dhp, Ep=Efp, NFp=Nfp)
    Wnb = mp['W_n'][din:]
    Wb0 = jnp.pad(Wnb[:dh], ((0, dhp - dh), (0, 0)))
    Wb1 = jnp.pad(Wnb[dh:], ((0, dhp - dh), (0, 0)))
    D = _tc_mm([(agg[:Nfp], Wb0), (agg[Nfp:], Wb1)], mp['b_n'],
               act=False, m_real=Nfr)
    gnode = _pad1(gn, Nfp, Ncr)
    x_new = _sc_node(T, gnode, D, S, dout=dout, NFp=Nfp)
    return x_new, [(e0, dh), (e1, dh)]


def kernel(z, edge_attr, params, edge_index2, edge_index1, edge_index0,
           m_id1, m_id0, e_idx1, e_idx0):
    p = params
    N2r, N1r, N0r = 2500, 5000, 10000
    E2r, E1r, E0r = 40000, 80000, 160000
    N2p, N1p, N0p = 2560, 5120, 10240
    E2p, E1p, E0p = 40960, 81920, 163840

    ei2 = edge_index2.astype(_I32)
    ei1 = edge_index1.astype(_I32)
    ei0 = edge_index0.astype(_I32)

    gn1 = jnp.full((N1r,), N2r, _I32).at[m_id1].set(jnp.arange(N2r, dtype=_I32))
    gn0 = jnp.full((N0r,), N1r, _I32).at[m_id0].set(jnp.arange(N1r, dtype=_I32))
    ge1 = jnp.full((E1r,), E2r, _I32).at[e_idx1].set(jnp.arange(E2r, dtype=_I32))
    ge0 = jnp.full((E0r,), E1r, _I32).at[e_idx0].set(jnp.arange(E1r, dtype=_I32))

    s2 = _sort_pkg(ei2[1], E2r, E2p, N2p)
    s1 = _sort_pkg(ei1[1], E1r, E1p, N1p)
    s0 = _sort_pkg(ei0[1], E0r, E0p, N0p)

    z2 = jnp.pad(z.reshape(-1, 1), ((0, 0), (0, 7)))
    W1z = jnp.pad(p['up_W1'], ((0, 7), (0, 0)))
    h = _tc_mm([(z2, W1z)], p['up_b1'], act=True, m_real=None, bm=128)
    haug = jnp.concatenate([h.T, jnp.ones((1, h.shape[0]), _F32)], axis=0)
    haug = jnp.pad(haug, ((0, 7), (0, 0)))
    W2a = jnp.concatenate([p['up_W2'].T, p['up_b2'][:, None]], axis=1)
    W2a = jnp.pad(W2a, ((0, N2p - N2r), (0, 7)))
    x = _tc_mm([(W2a, haug)], jnp.zeros((haug.shape[1],), _F32), act=False,
               m_real=N2r)

    e_list = [(_pad_rows(edge_attr, E2p), edge_attr.shape[1])]

    x, e_list = _plain_mpl(x, e_list, ei2[0], ei2[1], s2, p['bottom'], N2r,
                           N2p, E2r, E2p, CH=64)

    rp = p['r0']
    x_skip, _ = _gather_mpl(x, e_list, gn1, ge1, ei1[0], ei1[1], s1,
                            rp['skip'], N2r, N1r, N1p, E2r, E1r, E1p, CH=128)
    x1, e1_list = _plain_mpl(x, e_list, ei2[0], ei2[1], s2, rp['mpl1'], N2r,
                             N2p, E2r, E2p, CH=128)
    x, e_list = _gather_mpl(x1, e1_list, gn1, ge1, ei1[0], ei1[1], s1,
                            rp['mpl2'], N2r, N1r, N1p, E2r, E1r, E1p, CH=128,
                            S=x_skip)

    rp = p['r1']
    x_skip, _ = _gather_mpl(x, e_list, gn0, ge0, ei0[0], ei0[1], s0,
                            rp['skip'], N1r, N0r, N0p, E1r, E0r, E0p, CH=128)
    x1, e1_list = _plain_mpl(x, e_list, ei1[0], ei1[1], s1, rp['mpl1'], N1r,
                             N1p, E1r, E1p, CH=128)
    x, e_list = _gather_mpl(x1, e1_list, gn0, ge0, ei0[0], ei0[1], s0,
                            rp['mpl2'], N1r, N0r, N0p, E1r, E0r, E0p, CH=128,
                            S=x_skip)

    x, e_list = _plain_mpl(x, e_list, ei0[0], ei0[1], s0, p['final'], N0r,
                           N0p, E0r, E0p, CH=128)

    xn = _tc_head([(x, x.shape[1])], p['nd_W1'], p['nd_b1'], p['nd_W2'], p['nd_b2'],
                  p['nd_ln_g'], p['nd_ln_b'])
    en = _tc_head(e_list, p['ed_W1'], p['ed_b1'], p['ed_W2'], p['ed_b2'],
                  p['ed_ln_g'], p['ed_ln_b'])
    return xn[:N0r], en[:E0r]
